# baseline (speedup 1.0000x reference)
import jax
import jax.numpy as jnp
from jax import lax
from jax.experimental import pallas as pl
from jax.experimental.pallas import tpu as pltpu
from jax.experimental.pallas import tpu_sc as plsc

B = 16
C = 6
HW = 512 * 512
NV = 100000
NC = 2
NS = 16
NW = NC * NS
CH = 3200
NVP = NW * CH
VC = 128
NCH = CH // VC
D = 48


def _body(tf, tb, idxf, idxb, mf, mb, out,
          idxf_v, idxb_v, mf_v, mb_v, wf_v, wb_v, f_v, b_v, o_v, sem):
    cid = lax.axis_index("c")
    sid = lax.axis_index("s")
    wid = sid * NC + cid
    base = wid * CH

    pltpu.sync_copy(idxf.at[pl.ds(base, CH)], idxf_v)
    pltpu.sync_copy(idxb.at[pl.ds(base, CH)], idxb_v)
    pltpu.sync_copy(mf.at[pl.ds(base, CH)], mf_v)
    pltpu.sync_copy(mb.at[pl.ds(base, CH)], mb_v)

    def wloop(i, carry):
        s = pl.ds(i * 16, 16)
        a = lax.convert_element_type(mf_v[s], jnp.float32)
        bb = lax.convert_element_type(mb_v[s], jnp.float32)
        d = jnp.maximum(a + bb, 1.0)
        wf_v[s] = a / d
        wb_v[s] = bb / d
        return carry

    lax.fori_loop(0, CH // 16, wloop, 0)

    zeros = lax.shift_right_logical(lax.iota(jnp.int32, 16), 31)

    def chunk(k, carry):
        cb = k * VC
        cp_f = pltpu.async_copy(tf.at[idxf_v.at[pl.ds(cb, VC)]], f_v, sem)
        cp_b = pltpu.async_copy(tb.at[idxb_v.at[pl.ds(cb, VC)]], b_v, sem)
        cp_f.wait()
        cp_b.wait()

        def blend(r, cc):
            w1 = wf_v[pl.ds(cb + r, 16)].at[zeros].get(mode="promise_in_bounds")
            w2 = wb_v[pl.ds(cb + r, 16)].at[zeros].get(mode="promise_in_bounds")
            for j in range(3):
                s = pl.ds(r * D + j * 16, 16)
                o_v[s] = f_v[r, pl.ds(j * 16, 16)] * w1 + b_v[r, pl.ds(j * 16, 16)] * w2
            return cc

        lax.fori_loop(0, VC, blend, 0)

        pltpu.sync_copy(o_v, out.at[pl.ds((base + cb) * D, VC * D)])
        return carry

    lax.fori_loop(0, NCH, chunk, 0)


def _make_sample():
    return pl.kernel(
        _body,
        mesh=plsc.VectorSubcoreMesh(core_axis_name="c", subcore_axis_name="s"),
        compiler_params=pltpu.CompilerParams(use_tc_tiling_on_sc=False),
        out_type=jax.ShapeDtypeStruct((NVP * D,), jnp.float32),
        scratch_types=[
            pltpu.VMEM((CH,), jnp.int32),
            pltpu.VMEM((CH,), jnp.int32),
            pltpu.VMEM((CH,), jnp.int32),
            pltpu.VMEM((CH,), jnp.int32),
            pltpu.VMEM((CH + 16,), jnp.float32),
            pltpu.VMEM((CH + 16,), jnp.float32),
            pltpu.VMEM((VC, D), jnp.float32),
            pltpu.VMEM((VC, D), jnp.float32),
            pltpu.VMEM((VC * D,), jnp.float32),
            pltpu.SemaphoreType.DMA,
        ],
    )


def kernel(offset_imgs, vt_idx_front, vt_idx_back, mask_front, mask_back):
    tall = jnp.transpose(offset_imgs.reshape(B, C, HW), (2, 0, 1))
    tf = tall[:, :, :3].reshape(HW, D)
    tb = tall[:, :, 3:].reshape(HW, D)
    pad = NVP - NV
    idxf = jnp.pad(vt_idx_front, (0, pad))
    idxb = jnp.pad(vt_idx_back, (0, pad))
    mf = jnp.pad(mask_front, (0, pad))
    mb = jnp.pad(mask_back, (0, pad))
    o = _make_sample()(tf, tb, idxf, idxb, mf, mb)
    return jnp.transpose(o.reshape(NVP, B, 3), (1, 0, 2))[:, :NV, :]


# speedup vs baseline: 2.2129x; 2.2129x over previous
import jax
import jax.numpy as jnp
from jax import lax
from jax.experimental import pallas as pl
from jax.experimental.pallas import tpu as pltpu
from jax.experimental.pallas import tpu_sc as plsc

B = 16
C = 6
HW = 512 * 512
NV = 100000
NC = 2
NS = 16
NW = NC * NS
CH = 3200
NVP = NW * CH
VC = 128
NCH = CH // VC
D = 48
TD = 128


def _body(tab, idxf, idxb, mf, mb, out,
          idxf_v, idxb_v, mf_v, mb_v, wf_v, wb_v, f_v, b_v, o_v, sem):
    cid = lax.axis_index("c")
    sid = lax.axis_index("s")
    wid = sid * NC + cid
    base = wid * CH

    pltpu.sync_copy(idxf.at[pl.ds(base, CH)], idxf_v)
    pltpu.sync_copy(idxb.at[pl.ds(base, CH)], idxb_v)
    pltpu.sync_copy(mf.at[pl.ds(base, CH)], mf_v)
    pltpu.sync_copy(mb.at[pl.ds(base, CH)], mb_v)

    def wloop(i, carry):
        s = pl.ds(i * 16, 16)
        a = lax.convert_element_type(mf_v[s], jnp.float32)
        bb = lax.convert_element_type(mb_v[s], jnp.float32)
        d = jnp.maximum(a + bb, 1.0)
        wf_v[s] = a / d
        wb_v[s] = bb / d
        return carry

    lax.fori_loop(0, CH // 16, wloop, 0)

    zeros = lax.shift_right_logical(lax.iota(jnp.int32, 16), 31)

    def chunk(k, carry):
        cb = k * VC
        cp_f = pltpu.async_copy(tab.at[idxf_v.at[pl.ds(cb, VC)]], f_v, sem)
        cp_b = pltpu.async_copy(tab.at[idxb_v.at[pl.ds(cb, VC)]], b_v, sem)
        cp_f.wait()
        cp_b.wait()

        def blend(r, cc):
            w1 = wf_v[pl.ds(cb + r, 16)].at[zeros].get(mode="promise_in_bounds")
            w2 = wb_v[pl.ds(cb + r, 16)].at[zeros].get(mode="promise_in_bounds")
            for j in range(3):
                s = pl.ds(r * D + j * 16, 16)
                o_v[s] = (f_v[r, pl.ds(j * 16, 16)] * w1
                          + b_v[r, pl.ds(D + j * 16, 16)] * w2)
            return cc

        lax.fori_loop(0, VC, blend, 0)

        pltpu.sync_copy(o_v, out.at[pl.ds((base + cb) * D, VC * D)])
        return carry

    lax.fori_loop(0, NCH, chunk, 0)


def _make_sample():
    return pl.kernel(
        _body,
        mesh=plsc.VectorSubcoreMesh(core_axis_name="c", subcore_axis_name="s"),
        out_type=jax.ShapeDtypeStruct((NVP * D,), jnp.float32),
        scratch_types=[
            pltpu.VMEM((CH,), jnp.int32),
            pltpu.VMEM((CH,), jnp.int32),
            pltpu.VMEM((CH,), jnp.int32),
            pltpu.VMEM((CH,), jnp.int32),
            pltpu.VMEM((CH + 16,), jnp.float32),
            pltpu.VMEM((CH + 16,), jnp.float32),
            pltpu.VMEM((VC, TD), jnp.float32),
            pltpu.VMEM((VC, TD), jnp.float32),
            pltpu.VMEM((VC * D,), jnp.float32),
            pltpu.SemaphoreType.DMA,
        ],
    )


def kernel(offset_imgs, vt_idx_front, vt_idx_back, mask_front, mask_back):
    t = jnp.transpose(offset_imgs.reshape(B, C, HW), (2, 1, 0)).reshape(HW, C * B)
    tab = jnp.pad(t, ((0, 0), (0, TD - C * B)))
    pad = NVP - NV
    idxf = jnp.pad(vt_idx_front, (0, pad))
    idxb = jnp.pad(vt_idx_back, (0, pad))
    mf = jnp.pad(mask_front, (0, pad))
    mb = jnp.pad(mask_back, (0, pad))
    o = _make_sample()(tab, idxf, idxb, mf, mb)
    return jnp.transpose(o.reshape(NVP, 3, B), (2, 0, 1))[:, :NV, :]


# speedup vs baseline: 2.8608x; 1.2928x over previous
import jax
import jax.numpy as jnp
from jax import lax
from jax.experimental import pallas as pl
from jax.experimental.pallas import tpu as pltpu
from jax.experimental.pallas import tpu_sc as plsc

B = 16
C = 6
HW = 512 * 512
NV = 100000
NC = 2
NS = 16
NW = NC * NS
CH = 3200
NVP = NW * CH
VC = 64
NCH = CH // VC
D = 48
TD = 128


def _blend_chunk(o_v, f_v, b_v, wf_v, wb_v, cb, zeros):
    def blend(r, cc):
        w1 = wf_v[pl.ds(cb + r, 16)].at[zeros].get(mode="promise_in_bounds")
        w2 = wb_v[pl.ds(cb + r, 16)].at[zeros].get(mode="promise_in_bounds")
        for j in range(3):
            s = pl.ds(r * D + j * 16, 16)
            o_v[s] = (f_v[r, pl.ds(j * 16, 16)] * w1
                      + b_v[r, pl.ds(D + j * 16, 16)] * w2)
        return cc

    lax.fori_loop(0, VC, blend, 0)


def _body(tab, idxf, idxb, mf, mb, out,
          idxf_v, idxb_v, mf_v, mb_v, wf_v, wb_v,
          f0_v, b0_v, f1_v, b1_v, o_v,
          sf0, sb0, sf1, sb1):
    cid = lax.axis_index("c")
    sid = lax.axis_index("s")
    wid = sid * NC + cid
    base = wid * CH

    pltpu.sync_copy(idxf.at[pl.ds(base, CH)], idxf_v)
    pltpu.sync_copy(idxb.at[pl.ds(base, CH)], idxb_v)
    pltpu.sync_copy(mf.at[pl.ds(base, CH)], mf_v)
    pltpu.sync_copy(mb.at[pl.ds(base, CH)], mb_v)

    def wloop(i, carry):
        s = pl.ds(i * 16, 16)
        a = lax.convert_element_type(mf_v[s], jnp.float32)
        bb = lax.convert_element_type(mb_v[s], jnp.float32)
        d = jnp.maximum(a + bb, 1.0)
        wf_v[s] = a / d
        wb_v[s] = bb / d
        return carry

    lax.fori_loop(0, CH // 16, wloop, 0)

    zeros = lax.shift_right_logical(lax.iota(jnp.int32, 16), 31)
    bufs = ((f0_v, b0_v, sf0, sb0), (f1_v, b1_v, sf1, sb1))

    def issue(k, phase):
        fv, bv, sf, sb = bufs[phase]
        cb = k * VC
        pltpu.async_copy(tab.at[idxf_v.at[pl.ds(cb, VC)]], fv, sf)
        pltpu.async_copy(tab.at[idxb_v.at[pl.ds(cb, VC)]], bv, sb)

    issue(0, 0)
    issue(1, 1)

    def step(i, carry):
        for phase in range(2):
            k = i * 2 + phase
            fv, bv, sf, sb = bufs[phase]
            cb = k * VC
            pltpu.make_async_copy(tab.at[idxf_v.at[pl.ds(cb, VC)]], fv, sf).wait()
            pltpu.make_async_copy(tab.at[idxb_v.at[pl.ds(cb, VC)]], bv, sb).wait()
            _blend_chunk(o_v, fv, bv, wf_v, wb_v, cb, zeros)
            pltpu.sync_copy(o_v, out.at[pl.ds((base + cb) * D, VC * D)])

            @pl.when(k + 2 < NCH)
            def _():
                issue(k + 2, phase)

        return carry

    lax.fori_loop(0, NCH // 2, step, 0)


def _make_sample():
    return pl.kernel(
        _body,
        mesh=plsc.VectorSubcoreMesh(core_axis_name="c", subcore_axis_name="s"),
        out_type=jax.ShapeDtypeStruct((NVP * D,), jnp.float32),
        scratch_types=[
            pltpu.VMEM((CH,), jnp.int32),
            pltpu.VMEM((CH,), jnp.int32),
            pltpu.VMEM((CH,), jnp.int32),
            pltpu.VMEM((CH,), jnp.int32),
            pltpu.VMEM((CH + 16,), jnp.float32),
            pltpu.VMEM((CH + 16,), jnp.float32),
            pltpu.VMEM((VC, TD), jnp.float32),
            pltpu.VMEM((VC, TD), jnp.float32),
            pltpu.VMEM((VC, TD), jnp.float32),
            pltpu.VMEM((VC, TD), jnp.float32),
            pltpu.VMEM((VC * D,), jnp.float32),
            pltpu.SemaphoreType.DMA,
            pltpu.SemaphoreType.DMA,
            pltpu.SemaphoreType.DMA,
            pltpu.SemaphoreType.DMA,
        ],
    )


def kernel(offset_imgs, vt_idx_front, vt_idx_back, mask_front, mask_back):
    t = jnp.transpose(offset_imgs.reshape(B, C, HW), (2, 1, 0)).reshape(HW, C * B)
    tab = jnp.pad(t, ((0, 0), (0, TD - C * B)))
    pad = NVP - NV
    idxf = jnp.pad(vt_idx_front, (0, pad), mode="wrap")
    idxb = jnp.pad(vt_idx_back, (0, pad), mode="wrap")
    mf = jnp.pad(mask_front, (0, pad))
    mb = jnp.pad(mask_back, (0, pad))
    o = _make_sample()(tab, idxf, idxb, mf, mb)
    return jnp.transpose(o.reshape(NVP, 3, B), (2, 0, 1))[:, :NV, :]
